# Initial kernel scaffold; baseline (speedup 1.0000x reference)
import jax
import jax.numpy as jnp
from jax.experimental import pallas as pl


def kernel(encoding, pos, batch, graph_sizes, W_up, b_up, ln1_g, ln1_b, ln2_g, ln2_b, Wq, Wk, Wv, Wskip, We, lnm_g, lnm_b, W_mlp, b_mlp, mlp_g, mlp_b, W_out, b_out):
    raise NotImplementedError("write your pallas kernel here")


# speedup vs baseline: 1.8639x; 1.8639x over previous
import numpy as np
import jax
import jax.numpy as jnp
from jax.experimental import pallas as pl

B = 4
NPG = 256
N = B * NPG
D = 128
H = 4
DH = D // H
L = 2
R = 50
CUTOFF = 2.0
KNB = 100
NTYPES = 10

_CI = 32
_CR = 16
_HI = jax.lax.Precision.HIGHEST

_OFFS = np.linspace(0.0, CUTOFF, R).astype(np.float32)
_COEFF = float(-0.5 / (_OFFS[1] - _OFFS[0]) ** 2)
_SQRT_DH = float(np.sqrt(np.float32(DH)))


def _grid_positions_T():
    g = np.zeros((27, 3), dtype=np.float32)
    i = 0
    for xx in range(-1, 2):
        for yy in range(-1, 2):
            for zz in range(-1, 2):
                g[i] = (xx, yy, zz)
                i += 1
    return g.T.copy()


def _ln(x, g, b, eps=1e-5):
    m = x.mean(-1, keepdims=True)
    v = ((x - m) ** 2).mean(-1, keepdims=True)
    return (x - m) / jnp.sqrt(v + eps) * g + b


def _rowvec(v):
    n = v.shape[0]
    eye = (jax.lax.broadcasted_iota(jnp.int32, (n, n), 0)
           == jax.lax.broadcasted_iota(jnp.int32, (n, n), 1)).astype(jnp.float32)
    return jax.lax.dot_general(v, eye, (((0,), (0,)), ((), ())), precision=_HI)


def _fwd_kernel(enc_ref, pos_ref, gpT_ref, offs_ref, W2_ref, b2_ref,
                ln1g_ref, ln1b_ref, ln2g_ref, ln2b_ref,
                Wq_ref, Wk_ref, Wv_ref, Wsk_ref, We_ref,
                lnmg_ref, lnmb_ref, Wm_ref, bm_ref, mg_ref, mb_ref,
                Wo_ref, bo_ref, out_ref):
    f32 = jnp.float32

    enc = enc_ref[...].reshape(1, D)
    grid = (jnp.dot(enc, W2_ref[...], precision=_HI) + b2_ref[...]).reshape(27, D)
    grid = jax.nn.gelu(_ln(grid, ln1g_ref[...], ln1b_ref[...]))

    pos = pos_ref[...]
    gpT = gpT_ref[...]

    px, py, pz = pos[:, 0:1], pos[:, 1:2], pos[:, 2:3]
    d2g = ((px - gpT[0:1, :]) ** 2 + (py - gpT[1:2, :]) ** 2
           + (pz - gpT[2:3, :]) ** 2)
    jlt27 = (jax.lax.broadcasted_iota(jnp.int32, (27, 27), 0)
             < jax.lax.broadcasted_iota(jnp.int32, (27, 27), 1)).astype(f32)
    cmp27 = jnp.where(d2g[:, :, None] < d2g[:, None, :], 1.0,
                      jnp.where(d2g[:, :, None] == d2g[:, None, :],
                                jlt27[None], 0.0))
    rank27 = jnp.sum(cmp27, axis=1)
    w = jnp.where(rank27 < 3.0, 1.0 / jnp.clip(d2g, 1e-16, None), 0.0)
    x = jnp.dot(w, grid, precision=_HI) / jnp.sum(w, axis=1, keepdims=True)
    x = jax.nn.gelu(_ln(x, ln2g_ref[...], ln2b_ref[...]))

    offs3 = offs_ref[...].reshape(1, 1, R)
    jlt = (jax.lax.broadcasted_iota(jnp.int32, (NPG, NPG), 0)
           < jax.lax.broadcasted_iota(jnp.int32, (NPG, NPG), 1)).astype(f32)
    ridx = jax.lax.broadcasted_iota(jnp.int32, (NPG, NPG), 0)
    cidx = jax.lax.broadcasted_iota(jnp.int32, (NPG, NPG), 1)

    for l in range(L):
        px, py, pz = pos[:, 0:1], pos[:, 1:2], pos[:, 2:3]
        pd2 = ((px - _rowvec(px)) ** 2 + (py - _rowvec(py)) ** 2
               + (pz - _rowvec(pz)) ** 2)
        pd2 = jnp.where(ridx == cidx, pd2 + 1e9, pd2)

        selparts = []
        for ic in range(NPG // _CR):
            c = pd2[ic * _CR:(ic + 1) * _CR]
            a = c[:, :, None]
            bb = c[:, None, :]
            cmpv = jnp.where(a < bb, 1.0, jnp.where(a == bb, jlt[None], 0.0))
            rank = jnp.sum(cmpv, axis=1)
            selparts.append(jnp.where(
                (rank < float(KNB)) & (c < CUTOFF ** 2), 1.0, 0.0))
        sel = jnp.concatenate(selparts, axis=0)

        q = jnp.dot(x, Wq_ref[l], precision=_HI)
        kf = jnp.dot(x, Wk_ref[l], precision=_HI)
        vf = jnp.dot(x, Wv_ref[l], precision=_HI)
        sk = jnp.dot(x, Wsk_ref[l], precision=_HI)
        We_l = We_ref[l]

        chunks = []
        for ic in range(NPG // _CI):
            sl_q = slice(ic * _CI, (ic + 1) * _CI)
            q_c = q[sl_q]
            pd2_c = pd2[sl_q]
            sel_c = sel[sl_q]
            dist = jnp.sqrt(jnp.clip(pd2_c, 1e-12, None))
            rbf = jnp.exp(_COEFF * (dist[:, :, None] - offs3) ** 2)
            ee = jnp.dot(rbf.reshape(_CI * NPG, R), We_l,
                         precision=_HI).reshape(_CI, NPG, D)
            heads = []
            for h in range(H):
                hs = slice(h * DH, (h + 1) * DH)
                qh = q_c[:, hs]
                core = jax.lax.dot_general(
                    qh, kf[:, hs], (((1,), (1,)), ((), ())), precision=_HI)
                eterm = jnp.sum(qh[:, None, :] * ee[:, :, hs], axis=2)
                lg = (core + eterm) / _SQRT_DH
                lgm = jnp.where(sel_c > 0.0, lg, -1e9)
                m = jnp.max(lgm, axis=1, keepdims=True)
                p = jnp.exp(lgm - m) * sel_c
                den = jnp.sum(p, axis=1, keepdims=True)
                alpha = p / jnp.maximum(den, 1e-30)
                vh = jnp.dot(alpha, vf[:, hs], precision=_HI)
                evh = jnp.sum(alpha[:, :, None] * ee[:, :, hs], axis=1)
                heads.append(vh + evh)
            chunks.append(jnp.concatenate(heads, axis=1))
        attn = jnp.concatenate(chunks, axis=0)

        x = _ln(attn + sk, lnmg_ref[l], lnmb_ref[l])
        for f in range(2):
            x = jax.nn.gelu(_ln(
                jnp.dot(x, Wm_ref[l, f], precision=_HI) + bm_ref[l, f],
                mg_ref[l, f], mb_ref[l, f]))
        pos = pos + x[:, 0:3]
        x = jnp.concatenate([x[:, 0:D - 3], pos], axis=1)

    logits_out = jnp.dot(x, Wo_ref[...], precision=_HI) + bo_ref[...]
    out_ref[...] = jnp.concatenate([pos, logits_out], axis=1)


def kernel(encoding, pos, batch, graph_sizes, W_up, b_up, ln1_g, ln1_b,
           ln2_g, ln2_b, Wq, Wk, Wv, Wskip, We, lnm_g, lnm_b, W_mlp, b_mlp,
           mlp_g, mlp_b, W_out, b_out):
    W2 = W_up.reshape(D, D, 27).transpose(0, 2, 1).reshape(D, 27 * D)
    b2 = b_up.reshape(D, 27).transpose(1, 0).reshape(1, 27 * D)
    gpT = jnp.asarray(_grid_positions_T())
    offs = jnp.asarray(_OFFS.reshape(1, R))
    ln1g = ln1_g.reshape(1, D); ln1b = ln1_b.reshape(1, D)
    ln2g = ln2_g.reshape(1, D); ln2b = ln2_b.reshape(1, D)
    lnmg = lnm_g.reshape(L, 1, D); lnmb = lnm_b.reshape(L, 1, D)
    bm = b_mlp.reshape(L, 2, 1, D)
    mg = mlp_g.reshape(L, 2, 1, D); mb = mlp_b.reshape(L, 2, 1, D)
    bo = b_out.reshape(1, NTYPES)

    def full(a):
        nd = a.ndim
        return pl.BlockSpec(a.shape, lambda b, _n=nd: (0,) * _n)

    enc3 = encoding.reshape(B, 1, D)
    args = (enc3, pos, gpT, offs, W2, b2, ln1g, ln1b, ln2g, ln2b,
            Wq, Wk, Wv, Wskip, We, lnmg, lnmb, W_mlp, bm, mg, mb, W_out, bo)
    in_specs = [
        pl.BlockSpec((1, 1, D), lambda b: (b, 0, 0)),
        pl.BlockSpec((NPG, 3), lambda b: (b, 0)),
    ] + [full(a) for a in args[2:]]

    out = pl.pallas_call(
        _fwd_kernel,
        grid=(B,),
        in_specs=in_specs,
        out_specs=pl.BlockSpec((NPG, 3 + NTYPES), lambda b: (b, 0)),
        out_shape=jax.ShapeDtypeStruct((N, 3 + NTYPES), jnp.float32),
    )(*args)
    return out


# speedup vs baseline: 2.4050x; 1.2903x over previous
import numpy as np
import jax
import jax.numpy as jnp
from jax.experimental import pallas as pl
from jax.experimental.pallas import tpu as pltpu

B = 4
NPG = 256
N = B * NPG
D = 128
H = 4
DH = D // H
L = 2
R = 50
CUTOFF = 2.0
KNB = 100
NTYPES = 10

_CI = 32
_CR = 16
_HI = jax.lax.Precision.HIGHEST

_OFFS = np.linspace(0.0, CUTOFF, R).astype(np.float32)
_COEFF = float(-0.5 / (_OFFS[1] - _OFFS[0]) ** 2)
_SQRT_DH = float(np.sqrt(np.float32(DH)))


def _grid_positions_T():
    g = np.zeros((27, 3), dtype=np.float32)
    i = 0
    for xx in range(-1, 2):
        for yy in range(-1, 2):
            for zz in range(-1, 2):
                g[i] = (xx, yy, zz)
                i += 1
    return g.T.copy()


def _ln(x, g, b, eps=1e-5):
    m = x.mean(-1, keepdims=True)
    v = ((x - m) ** 2).mean(-1, keepdims=True)
    return (x - m) / jnp.sqrt(v + eps) * g + b


def _rowvec(v):
    n = v.shape[0]
    eye = (jax.lax.broadcasted_iota(jnp.int32, (n, n), 0)
           == jax.lax.broadcasted_iota(jnp.int32, (n, n), 1)).astype(jnp.float32)
    return jax.lax.dot_general(v, eye, (((0,), (0,)), ((), ())), precision=_HI)


def _fwd_kernel(enc_ref, pos_ref, gpT_ref, offs_ref, W2_ref, b2_ref,
                ln1g_ref, ln1b_ref, ln2g_ref, ln2b_ref,
                Wq_ref, Wk_ref, Wv_ref, Wsk_ref, We_ref,
                lnmg_ref, lnmb_ref, Wm_ref, bm_ref, mg_ref, mb_ref,
                Wo_ref, bo_ref, out_ref):
    f32 = jnp.float32

    enc = enc_ref[...].reshape(1, D)
    grid = (jnp.dot(enc, W2_ref[...], precision=_HI) + b2_ref[...]).reshape(27, D)
    grid = jax.nn.gelu(_ln(grid, ln1g_ref[...], ln1b_ref[...]))

    pos = pos_ref[...]
    gpT = gpT_ref[...]

    px, py, pz = pos[:, 0:1], pos[:, 1:2], pos[:, 2:3]
    d2g = ((px - gpT[0:1, :]) ** 2 + (py - gpT[1:2, :]) ** 2
           + (pz - gpT[2:3, :]) ** 2)
    jlt27 = (jax.lax.broadcasted_iota(jnp.int32, (27, 27), 0)
             < jax.lax.broadcasted_iota(jnp.int32, (27, 27), 1)).astype(f32)
    cmp27 = jnp.where(d2g[:, :, None] < d2g[:, None, :], 1.0,
                      jnp.where(d2g[:, :, None] == d2g[:, None, :],
                                jlt27[None], 0.0))
    rank27 = jnp.sum(cmp27, axis=1)
    w = jnp.where(rank27 < 3.0, 1.0 / jnp.clip(d2g, 1e-16, None), 0.0)
    x = jnp.dot(w, grid, precision=_HI) / jnp.sum(w, axis=1, keepdims=True)
    x = jax.nn.gelu(_ln(x, ln2g_ref[...], ln2b_ref[...]))

    offs3 = offs_ref[...].reshape(1, 1, R)
    jlt = (jax.lax.broadcasted_iota(jnp.int32, (NPG, NPG), 0)
           < jax.lax.broadcasted_iota(jnp.int32, (NPG, NPG), 1)).astype(f32)
    ridx = jax.lax.broadcasted_iota(jnp.int32, (NPG, NPG), 0)
    cidx = jax.lax.broadcasted_iota(jnp.int32, (NPG, NPG), 1)

    for l in range(L):
        px, py, pz = pos[:, 0:1], pos[:, 1:2], pos[:, 2:3]
        pd2 = ((px - _rowvec(px)) ** 2 + (py - _rowvec(py)) ** 2
               + (pz - _rowvec(pz)) ** 2)
        pd2 = jnp.where(ridx == cidx, pd2 + 1e9, pd2)

        selparts = []
        for ic in range(NPG // _CR):
            c = pd2[ic * _CR:(ic + 1) * _CR]
            a = c[:, :, None]
            bb = c[:, None, :]
            cmpv = jnp.where(a < bb, 1.0, jnp.where(a == bb, jlt[None], 0.0))
            rank = jnp.sum(cmpv, axis=1)
            selparts.append(jnp.where(
                (rank < float(KNB)) & (c < CUTOFF ** 2), 1.0, 0.0))
        sel = jnp.concatenate(selparts, axis=0)

        q = jnp.dot(x, Wq_ref[l], precision=_HI)
        kf = jnp.dot(x, Wk_ref[l], precision=_HI)
        vf = jnp.dot(x, Wv_ref[l], precision=_HI)
        sk = jnp.dot(x, Wsk_ref[l], precision=_HI)
        We_l = We_ref[l]

        chunks = []
        for ic in range(NPG // _CI):
            sl_q = slice(ic * _CI, (ic + 1) * _CI)
            q_c = q[sl_q]
            pd2_c = pd2[sl_q]
            sel_c = sel[sl_q]
            dist = jnp.sqrt(jnp.clip(pd2_c, 1e-12, None))
            rbf = jnp.exp(_COEFF * (dist[:, :, None] - offs3) ** 2)
            ee = jnp.dot(rbf.reshape(_CI * NPG, R), We_l,
                         precision=jax.lax.Precision.DEFAULT).reshape(_CI, NPG, D)
            heads = []
            for h in range(H):
                hs = slice(h * DH, (h + 1) * DH)
                qh = q_c[:, hs]
                core = jax.lax.dot_general(
                    qh, kf[:, hs], (((1,), (1,)), ((), ())), precision=_HI)
                eterm = jnp.sum(qh[:, None, :] * ee[:, :, hs], axis=2)
                lg = (core + eterm) / _SQRT_DH
                lgm = jnp.where(sel_c > 0.0, lg, -1e9)
                m = jnp.max(lgm, axis=1, keepdims=True)
                p = jnp.exp(lgm - m) * sel_c
                den = jnp.sum(p, axis=1, keepdims=True)
                alpha = p / jnp.maximum(den, 1e-30)
                vh = jnp.dot(alpha, vf[:, hs], precision=_HI)
                evh = jnp.sum(alpha[:, :, None] * ee[:, :, hs], axis=1)
                heads.append(vh + evh)
            chunks.append(jnp.concatenate(heads, axis=1))
        attn = jnp.concatenate(chunks, axis=0)

        x = _ln(attn + sk, lnmg_ref[l], lnmb_ref[l])
        for f in range(2):
            x = jax.nn.gelu(_ln(
                jnp.dot(x, Wm_ref[l, f], precision=_HI) + bm_ref[l, f],
                mg_ref[l, f], mb_ref[l, f]))
        pos = pos + x[:, 0:3]
        x = jnp.concatenate([x[:, 0:D - 3], pos], axis=1)

    logits_out = jnp.dot(x, Wo_ref[...], precision=_HI) + bo_ref[...]
    out_ref[...] = jnp.concatenate([pos, logits_out], axis=1)


def kernel(encoding, pos, batch, graph_sizes, W_up, b_up, ln1_g, ln1_b,
           ln2_g, ln2_b, Wq, Wk, Wv, Wskip, We, lnm_g, lnm_b, W_mlp, b_mlp,
           mlp_g, mlp_b, W_out, b_out):
    W2 = W_up.reshape(D, D, 27).transpose(0, 2, 1).reshape(D, 27 * D)
    b2 = b_up.reshape(D, 27).transpose(1, 0).reshape(1, 27 * D)
    gpT = jnp.asarray(_grid_positions_T())
    offs = jnp.asarray(_OFFS.reshape(1, R))
    ln1g = ln1_g.reshape(1, D); ln1b = ln1_b.reshape(1, D)
    ln2g = ln2_g.reshape(1, D); ln2b = ln2_b.reshape(1, D)
    lnmg = lnm_g.reshape(L, 1, D); lnmb = lnm_b.reshape(L, 1, D)
    bm = b_mlp.reshape(L, 2, 1, D)
    mg = mlp_g.reshape(L, 2, 1, D); mb = mlp_b.reshape(L, 2, 1, D)
    bo = b_out.reshape(1, NTYPES)

    def full(a):
        nd = a.ndim
        return pl.BlockSpec(a.shape, lambda b, _n=nd: (0,) * _n)

    enc3 = encoding.reshape(B, 1, D)
    args = (enc3, pos, gpT, offs, W2, b2, ln1g, ln1b, ln2g, ln2b,
            Wq, Wk, Wv, Wskip, We, lnmg, lnmb, W_mlp, bm, mg, mb, W_out, bo)
    in_specs = [
        pl.BlockSpec((1, 1, D), lambda b: (b, 0, 0)),
        pl.BlockSpec((NPG, 3), lambda b: (b, 0)),
    ] + [full(a) for a in args[2:]]

    out = pl.pallas_call(
        _fwd_kernel,
        grid=(B,),
        in_specs=in_specs,
        out_specs=pl.BlockSpec((NPG, 3 + NTYPES), lambda b: (b, 0)),
        out_shape=jax.ShapeDtypeStruct((N, 3 + NTYPES), jnp.float32),
        compiler_params=pltpu.CompilerParams(
            dimension_semantics=("parallel",)),
    )(*args)
    return out


# speedup vs baseline: 2.5333x; 1.0533x over previous
import numpy as np
import jax
import jax.numpy as jnp
from jax.experimental import pallas as pl
from jax.experimental.pallas import tpu as pltpu

B = 4
NPG = 256
N = B * NPG
D = 128
H = 4
DH = D // H
L = 2
R = 50
CUTOFF = 2.0
KNB = 100
NTYPES = 10

_CI = 32
_CR = 16
_HI = jax.lax.Precision.HIGHEST

_OFFS = np.linspace(0.0, CUTOFF, R).astype(np.float32)
_COEFF = float(-0.5 / (_OFFS[1] - _OFFS[0]) ** 2)
_SQRT_DH = float(np.sqrt(np.float32(DH)))


def _grid_positions_T():
    g = np.zeros((27, 3), dtype=np.float32)
    i = 0
    for xx in range(-1, 2):
        for yy in range(-1, 2):
            for zz in range(-1, 2):
                g[i] = (xx, yy, zz)
                i += 1
    return g.T.copy()


def _ln(x, g, b, eps=1e-5):
    m = x.mean(-1, keepdims=True)
    v = ((x - m) ** 2).mean(-1, keepdims=True)
    return (x - m) / jnp.sqrt(v + eps) * g + b


def _rowvec(v):
    n = v.shape[0]
    eye = (jax.lax.broadcasted_iota(jnp.int32, (n, n), 0)
           == jax.lax.broadcasted_iota(jnp.int32, (n, n), 1)).astype(jnp.float32)
    return jax.lax.dot_general(v, eye, (((0,), (0,)), ((), ())), precision=_HI)


def _fwd_kernel(enc_ref, pos_ref, gpT_ref, offs_ref, W2_ref, b2_ref,
                ln1g_ref, ln1b_ref, ln2g_ref, ln2b_ref,
                Wq_ref, Wk_ref, Wv_ref, Wsk_ref, We_ref,
                lnmg_ref, lnmb_ref, Wm_ref, bm_ref, mg_ref, mb_ref,
                Wo_ref, bo_ref, out_ref):
    f32 = jnp.float32

    enc = enc_ref[...].reshape(1, D)
    grid = (jnp.dot(enc, W2_ref[...], precision=_HI) + b2_ref[...]).reshape(27, D)
    grid = jax.nn.gelu(_ln(grid, ln1g_ref[...], ln1b_ref[...]))

    pos = pos_ref[...]
    gpT = gpT_ref[...]

    px, py, pz = pos[:, 0:1], pos[:, 1:2], pos[:, 2:3]
    d2g = ((px - gpT[0:1, :]) ** 2 + (py - gpT[1:2, :]) ** 2
           + (pz - gpT[2:3, :]) ** 2)
    jlt27 = (jax.lax.broadcasted_iota(jnp.int32, (27, 27), 0)
             < jax.lax.broadcasted_iota(jnp.int32, (27, 27), 1)).astype(f32)
    cmp27 = jnp.where(d2g[:, :, None] < d2g[:, None, :], 1.0,
                      jnp.where(d2g[:, :, None] == d2g[:, None, :],
                                jlt27[None], 0.0))
    rank27 = jnp.sum(cmp27, axis=1)
    w = jnp.where(rank27 < 3.0, 1.0 / jnp.clip(d2g, 1e-16, None), 0.0)
    x = jnp.dot(w, grid, precision=_HI) / jnp.sum(w, axis=1, keepdims=True)
    x = jax.nn.gelu(_ln(x, ln2g_ref[...], ln2b_ref[...]))

    offs3 = offs_ref[...].reshape(1, 1, R)
    jlt = (jax.lax.broadcasted_iota(jnp.int32, (NPG, NPG), 0)
           < jax.lax.broadcasted_iota(jnp.int32, (NPG, NPG), 1)).astype(f32)
    ridx = jax.lax.broadcasted_iota(jnp.int32, (NPG, NPG), 0)
    cidx = jax.lax.broadcasted_iota(jnp.int32, (NPG, NPG), 1)

    for l in range(L):
        px, py, pz = pos[:, 0:1], pos[:, 1:2], pos[:, 2:3]
        pd2 = ((px - _rowvec(px)) ** 2 + (py - _rowvec(py)) ** 2
               + (pz - _rowvec(pz)) ** 2)
        pd2 = jnp.where(ridx == cidx, pd2 + 1e9, pd2)

        keys = jax.lax.bitcast_convert_type(pd2, jnp.int32)

        def _bs_body(_, lohi):
            lo, hi = lohi
            mid = lo + (hi - lo) // 2
            cle = jnp.sum((keys <= mid).astype(jnp.int32), axis=1,
                          keepdims=True)
            ge = cle >= KNB
            return (jnp.where(ge, lo, mid + 1), jnp.where(ge, mid, hi))

        lo0 = jnp.zeros((NPG, 1), jnp.int32)
        hi0 = jnp.full((NPG, 1), 0x7f800000, jnp.int32)
        tau, _ = jax.lax.fori_loop(0, 31, _bs_body, (lo0, hi0))
        clt = jnp.sum((keys < tau).astype(jnp.int32), axis=1, keepdims=True)
        need = (KNB - clt).astype(jnp.float32)
        eqf = (keys == tau).astype(jnp.float32)
        eqrank = jnp.dot(eqf, jlt, precision=jax.lax.Precision.DEFAULT)
        sel = jnp.where(((keys < tau) | ((eqf > 0.0) & (eqrank < need)))
                        & (pd2 < CUTOFF ** 2), 1.0, 0.0)

        q = jnp.dot(x, Wq_ref[l], precision=_HI)
        kf = jnp.dot(x, Wk_ref[l], precision=_HI)
        vf = jnp.dot(x, Wv_ref[l], precision=_HI)
        sk = jnp.dot(x, Wsk_ref[l], precision=_HI)
        We_l = We_ref[l]

        chunks = []
        for ic in range(NPG // _CI):
            sl_q = slice(ic * _CI, (ic + 1) * _CI)
            q_c = q[sl_q]
            pd2_c = pd2[sl_q]
            sel_c = sel[sl_q]
            dist = jnp.sqrt(jnp.clip(pd2_c, 1e-12, None))
            rbf = jnp.exp(_COEFF * (dist[:, :, None] - offs3) ** 2)
            ee = jnp.dot(rbf.reshape(_CI * NPG, R), We_l,
                         precision=jax.lax.Precision.DEFAULT).reshape(_CI, NPG, D)
            heads = []
            for h in range(H):
                hs = slice(h * DH, (h + 1) * DH)
                qh = q_c[:, hs]
                core = jax.lax.dot_general(
                    qh, kf[:, hs], (((1,), (1,)), ((), ())), precision=_HI)
                eterm = jnp.sum(qh[:, None, :] * ee[:, :, hs], axis=2)
                lg = (core + eterm) / _SQRT_DH
                lgm = jnp.where(sel_c > 0.0, lg, -1e9)
                m = jnp.max(lgm, axis=1, keepdims=True)
                p = jnp.exp(lgm - m) * sel_c
                den = jnp.sum(p, axis=1, keepdims=True)
                alpha = p / jnp.maximum(den, 1e-30)
                vh = jnp.dot(alpha, vf[:, hs], precision=_HI)
                evh = jnp.sum(alpha[:, :, None] * ee[:, :, hs], axis=1)
                heads.append(vh + evh)
            chunks.append(jnp.concatenate(heads, axis=1))
        attn = jnp.concatenate(chunks, axis=0)

        x = _ln(attn + sk, lnmg_ref[l], lnmb_ref[l])
        for f in range(2):
            x = jax.nn.gelu(_ln(
                jnp.dot(x, Wm_ref[l, f], precision=_HI) + bm_ref[l, f],
                mg_ref[l, f], mb_ref[l, f]))
        pos = pos + x[:, 0:3]
        x = jnp.concatenate([x[:, 0:D - 3], pos], axis=1)

    logits_out = jnp.dot(x, Wo_ref[...], precision=_HI) + bo_ref[...]
    out_ref[...] = jnp.concatenate([pos, logits_out], axis=1)


def kernel(encoding, pos, batch, graph_sizes, W_up, b_up, ln1_g, ln1_b,
           ln2_g, ln2_b, Wq, Wk, Wv, Wskip, We, lnm_g, lnm_b, W_mlp, b_mlp,
           mlp_g, mlp_b, W_out, b_out):
    W2 = W_up.reshape(D, D, 27).transpose(0, 2, 1).reshape(D, 27 * D)
    b2 = b_up.reshape(D, 27).transpose(1, 0).reshape(1, 27 * D)
    gpT = jnp.asarray(_grid_positions_T())
    offs = jnp.asarray(_OFFS.reshape(1, R))
    ln1g = ln1_g.reshape(1, D); ln1b = ln1_b.reshape(1, D)
    ln2g = ln2_g.reshape(1, D); ln2b = ln2_b.reshape(1, D)
    lnmg = lnm_g.reshape(L, 1, D); lnmb = lnm_b.reshape(L, 1, D)
    bm = b_mlp.reshape(L, 2, 1, D)
    mg = mlp_g.reshape(L, 2, 1, D); mb = mlp_b.reshape(L, 2, 1, D)
    bo = b_out.reshape(1, NTYPES)

    def full(a):
        nd = a.ndim
        return pl.BlockSpec(a.shape, lambda b, _n=nd: (0,) * _n)

    enc3 = encoding.reshape(B, 1, D)
    args = (enc3, pos, gpT, offs, W2, b2, ln1g, ln1b, ln2g, ln2b,
            Wq, Wk, Wv, Wskip, We, lnmg, lnmb, W_mlp, bm, mg, mb, W_out, bo)
    in_specs = [
        pl.BlockSpec((1, 1, D), lambda b: (b, 0, 0)),
        pl.BlockSpec((NPG, 3), lambda b: (b, 0)),
    ] + [full(a) for a in args[2:]]

    out = pl.pallas_call(
        _fwd_kernel,
        grid=(B,),
        in_specs=in_specs,
        out_specs=pl.BlockSpec((NPG, 3 + NTYPES), lambda b: (b, 0)),
        out_shape=jax.ShapeDtypeStruct((N, 3 + NTYPES), jnp.float32),
        compiler_params=pltpu.CompilerParams(
            dimension_semantics=("parallel",)),
    )(*args)
    return out


# speedup vs baseline: 2.7969x; 1.1041x over previous
import numpy as np
import jax
import jax.numpy as jnp
from jax.experimental import pallas as pl
from jax.experimental.pallas import tpu as pltpu

B = 4
NPG = 256
N = B * NPG
D = 128
H = 4
DH = D // H
L = 2
R = 50
CUTOFF = 2.0
KNB = 100
NTYPES = 10

_CI = 64
_HI = jax.lax.Precision.HIGHEST
_DEF = jax.lax.Precision.DEFAULT


def _dot3(a, b, dims=None):
    if dims is None:
        dims = (((a.ndim - 1,), (0,)), ((), ()))
    ah = a.astype(jnp.bfloat16).astype(jnp.float32)
    al = a - ah
    bh = b.astype(jnp.bfloat16).astype(jnp.float32)
    bl = b - bh

    def d(x, y):
        return jax.lax.dot_general(x, y, dims, precision=_DEF)

    return d(ah, bh) + d(al, bh) + d(ah, bl)

_OFFS = np.linspace(0.0, CUTOFF, R).astype(np.float32)
_COEFF = float(-0.5 / (_OFFS[1] - _OFFS[0]) ** 2)
_SQRT_DH = float(np.sqrt(np.float32(DH)))


def _grid_positions_T():
    g = np.zeros((27, 3), dtype=np.float32)
    i = 0
    for xx in range(-1, 2):
        for yy in range(-1, 2):
            for zz in range(-1, 2):
                g[i] = (xx, yy, zz)
                i += 1
    return g.T.copy()


def _ln(x, g, b, eps=1e-5):
    m = x.mean(-1, keepdims=True)
    v = ((x - m) ** 2).mean(-1, keepdims=True)
    return (x - m) / jnp.sqrt(v + eps) * g + b


def _rowvec(v):
    n = v.shape[0]
    eye = (jax.lax.broadcasted_iota(jnp.int32, (n, n), 0)
           == jax.lax.broadcasted_iota(jnp.int32, (n, n), 1)).astype(jnp.float32)
    return jax.lax.dot_general(v, eye, (((0,), (0,)), ((), ())), precision=_HI)


def _fwd_kernel(enc_ref, pos_ref, gpT_ref, offs_ref, W2_ref, b2_ref,
                ln1g_ref, ln1b_ref, ln2g_ref, ln2b_ref,
                Wq_ref, Wk_ref, Wv_ref, Wsk_ref, We_ref,
                lnmg_ref, lnmb_ref, Wm_ref, bm_ref, mg_ref, mb_ref,
                Wo_ref, bo_ref, out_ref):
    f32 = jnp.float32

    enc = enc_ref[...].reshape(1, D)
    grid = (_dot3(enc, W2_ref[...]) + b2_ref[...]).reshape(27, D)
    grid = jax.nn.gelu(_ln(grid, ln1g_ref[...], ln1b_ref[...]))

    pos = pos_ref[...]
    gpT = gpT_ref[...]

    px, py, pz = pos[:, 0:1], pos[:, 1:2], pos[:, 2:3]
    d2g = ((px - gpT[0:1, :]) ** 2 + (py - gpT[1:2, :]) ** 2
           + (pz - gpT[2:3, :]) ** 2)
    jlt27 = (jax.lax.broadcasted_iota(jnp.int32, (27, 27), 0)
             < jax.lax.broadcasted_iota(jnp.int32, (27, 27), 1)).astype(f32)
    cmp27 = jnp.where(d2g[:, :, None] < d2g[:, None, :], 1.0,
                      jnp.where(d2g[:, :, None] == d2g[:, None, :],
                                jlt27[None], 0.0))
    rank27 = jnp.sum(cmp27, axis=1)
    w = jnp.where(rank27 < 3.0, 1.0 / jnp.clip(d2g, 1e-16, None), 0.0)
    x = _dot3(w, grid) / jnp.sum(w, axis=1, keepdims=True)
    x = jax.nn.gelu(_ln(x, ln2g_ref[...], ln2b_ref[...]))

    offs3 = offs_ref[...].reshape(1, 1, R)
    jlt = (jax.lax.broadcasted_iota(jnp.int32, (NPG, NPG), 0)
           < jax.lax.broadcasted_iota(jnp.int32, (NPG, NPG), 1)).astype(f32)
    ridx = jax.lax.broadcasted_iota(jnp.int32, (NPG, NPG), 0)
    cidx = jax.lax.broadcasted_iota(jnp.int32, (NPG, NPG), 1)

    for l in range(L):
        px, py, pz = pos[:, 0:1], pos[:, 1:2], pos[:, 2:3]
        pd2 = ((px - _rowvec(px)) ** 2 + (py - _rowvec(py)) ** 2
               + (pz - _rowvec(pz)) ** 2)
        pd2 = jnp.where(ridx == cidx, pd2 + 1e9, pd2)

        keys = jax.lax.bitcast_convert_type(pd2, jnp.int32)

        def _bs_body(_, lohi):
            lo, hi = lohi
            mid = lo + (hi - lo) // 2
            cle = jnp.sum((keys <= mid).astype(jnp.int32), axis=1,
                          keepdims=True)
            ge = cle >= KNB
            return (jnp.where(ge, lo, mid + 1), jnp.where(ge, mid, hi))

        lo0 = jnp.zeros((NPG, 1), jnp.int32)
        hi0 = jnp.full((NPG, 1), 0x7f800000, jnp.int32)
        tau, _ = jax.lax.fori_loop(0, 31, _bs_body, (lo0, hi0))
        clt = jnp.sum((keys < tau).astype(jnp.int32), axis=1, keepdims=True)
        need = (KNB - clt).astype(jnp.float32)
        eqf = (keys == tau).astype(jnp.float32)
        eqrank = jnp.dot(eqf, jlt, precision=jax.lax.Precision.DEFAULT)
        sel = jnp.where(((keys < tau) | ((eqf > 0.0) & (eqrank < need)))
                        & (pd2 < CUTOFF ** 2), 1.0, 0.0)

        q = _dot3(x, Wq_ref[l])
        kf = _dot3(x, Wk_ref[l])
        vf = _dot3(x, Wv_ref[l])
        sk = _dot3(x, Wsk_ref[l])
        We_l = We_ref[l]

        chunks = []
        for ic in range(NPG // _CI):
            sl_q = slice(ic * _CI, (ic + 1) * _CI)
            q_c = q[sl_q]
            pd2_c = pd2[sl_q]
            sel_c = sel[sl_q]
            dist = jnp.sqrt(jnp.clip(pd2_c, 1e-12, None))
            rbf = jnp.exp(_COEFF * (dist[:, :, None] - offs3) ** 2)
            ee = jax.lax.dot_general(
                rbf.reshape(_CI * NPG, R), We_l, (((1,), (0,)), ((), ())),
                precision=_DEF).reshape(_CI, NPG, D)
            heads = []
            for h in range(H):
                hs = slice(h * DH, (h + 1) * DH)
                qh = q_c[:, hs]
                core = _dot3(qh, kf[:, hs], (((1,), (1,)), ((), ())))
                eterm = jnp.sum(qh[:, None, :] * ee[:, :, hs], axis=2)
                lg = (core + eterm) / _SQRT_DH
                lgm = jnp.where(sel_c > 0.0, lg, -1e9)
                m = jnp.max(lgm, axis=1, keepdims=True)
                p = jnp.exp(lgm - m) * sel_c
                den = jnp.sum(p, axis=1, keepdims=True)
                alpha = p / jnp.maximum(den, 1e-30)
                vh = _dot3(alpha, vf[:, hs])
                evh = jnp.sum(alpha[:, :, None] * ee[:, :, hs], axis=1)
                heads.append(vh + evh)
            chunks.append(jnp.concatenate(heads, axis=1))
        attn = jnp.concatenate(chunks, axis=0)

        x = _ln(attn + sk, lnmg_ref[l], lnmb_ref[l])
        for f in range(2):
            x = jax.nn.gelu(_ln(
                _dot3(x, Wm_ref[l, f]) + bm_ref[l, f],
                mg_ref[l, f], mb_ref[l, f]))
        pos = pos + x[:, 0:3]
        x = jnp.concatenate([x[:, 0:D - 3], pos], axis=1)

    logits_out = _dot3(x, Wo_ref[...]) + bo_ref[...]
    out_ref[...] = jnp.concatenate([pos, logits_out], axis=1)


def kernel(encoding, pos, batch, graph_sizes, W_up, b_up, ln1_g, ln1_b,
           ln2_g, ln2_b, Wq, Wk, Wv, Wskip, We, lnm_g, lnm_b, W_mlp, b_mlp,
           mlp_g, mlp_b, W_out, b_out):
    W2 = W_up.reshape(D, D, 27).transpose(0, 2, 1).reshape(D, 27 * D)
    b2 = b_up.reshape(D, 27).transpose(1, 0).reshape(1, 27 * D)
    gpT = jnp.asarray(_grid_positions_T())
    offs = jnp.asarray(_OFFS.reshape(1, R))
    ln1g = ln1_g.reshape(1, D); ln1b = ln1_b.reshape(1, D)
    ln2g = ln2_g.reshape(1, D); ln2b = ln2_b.reshape(1, D)
    lnmg = lnm_g.reshape(L, 1, D); lnmb = lnm_b.reshape(L, 1, D)
    bm = b_mlp.reshape(L, 2, 1, D)
    mg = mlp_g.reshape(L, 2, 1, D); mb = mlp_b.reshape(L, 2, 1, D)
    bo = b_out.reshape(1, NTYPES)

    def full(a):
        nd = a.ndim
        return pl.BlockSpec(a.shape, lambda b, _n=nd: (0,) * _n)

    enc3 = encoding.reshape(B, 1, D)
    args = (enc3, pos, gpT, offs, W2, b2, ln1g, ln1b, ln2g, ln2b,
            Wq, Wk, Wv, Wskip, We, lnmg, lnmb, W_mlp, bm, mg, mb, W_out, bo)
    in_specs = [
        pl.BlockSpec((1, 1, D), lambda b: (b, 0, 0)),
        pl.BlockSpec((NPG, 3), lambda b: (b, 0)),
    ] + [full(a) for a in args[2:]]

    out = pl.pallas_call(
        _fwd_kernel,
        grid=(B,),
        in_specs=in_specs,
        out_specs=pl.BlockSpec((NPG, 3 + NTYPES), lambda b: (b, 0)),
        out_shape=jax.ShapeDtypeStruct((N, 3 + NTYPES), jnp.float32),
        compiler_params=pltpu.CompilerParams(
            dimension_semantics=("parallel",)),
    )(*args)
    return out


# speedup vs baseline: 3.4304x; 1.2265x over previous
import numpy as np
import jax
import jax.numpy as jnp
from jax.experimental import pallas as pl
from jax.experimental.pallas import tpu as pltpu

B = 4
NPG = 256
N = B * NPG
D = 128
H = 4
DH = D // H
L = 2
R = 50
CUTOFF = 2.0
KNB = 100
NTYPES = 10

_CI = 64
_HI = jax.lax.Precision.HIGHEST
_DEF = jax.lax.Precision.DEFAULT


def _dot3(a, b, dims=None):
    if dims is None:
        dims = (((a.ndim - 1,), (0,)), ((), ()))
    ah = a.astype(jnp.bfloat16).astype(jnp.float32)
    al = a - ah
    bh = b.astype(jnp.bfloat16).astype(jnp.float32)
    bl = b - bh

    def d(x, y):
        return jax.lax.dot_general(x, y, dims, precision=_DEF)

    return d(ah, bh) + d(al, bh) + d(ah, bl)

_OFFS = np.linspace(0.0, CUTOFF, R).astype(np.float32)
_COEFF = float(-0.5 / (_OFFS[1] - _OFFS[0]) ** 2)
_SQRT_DH = float(np.sqrt(np.float32(DH)))
_LOG2E = float(np.log2(np.exp(np.float64(1.0))))


def _grid_positions_T():
    g = np.zeros((27, 3), dtype=np.float32)
    i = 0
    for xx in range(-1, 2):
        for yy in range(-1, 2):
            for zz in range(-1, 2):
                g[i] = (xx, yy, zz)
                i += 1
    return g.T.copy()


def _ln(x, g, b, eps=1e-5):
    m = x.mean(-1, keepdims=True)
    v = ((x - m) ** 2).mean(-1, keepdims=True)
    return (x - m) / jnp.sqrt(v + eps) * g + b


def _rowvec(v):
    n = v.shape[0]
    eye = (jax.lax.broadcasted_iota(jnp.int32, (n, n), 0)
           == jax.lax.broadcasted_iota(jnp.int32, (n, n), 1)).astype(jnp.float32)
    return jax.lax.dot_general(v, eye, (((0,), (0,)), ((), ())), precision=_HI)


def _fwd_kernel(enc_ref, pos_ref, gpT_ref, offs_ref, W2_ref, b2_ref,
                ln1g_ref, ln1b_ref, ln2g_ref, ln2b_ref,
                Wq_ref, Wk_ref, Wv_ref, Wsk_ref, We_ref,
                lnmg_ref, lnmb_ref, Wm_ref, bm_ref, mg_ref, mb_ref,
                Wo_ref, bo_ref, out_ref):
    f32 = jnp.float32

    enc = enc_ref[...].reshape(1, D)
    grid = (_dot3(enc, W2_ref[...]) + b2_ref[...]).reshape(27, D)
    grid = jax.nn.gelu(_ln(grid, ln1g_ref[...], ln1b_ref[...]))

    pos = pos_ref[...]
    gpT = gpT_ref[...]

    px, py, pz = pos[:, 0:1], pos[:, 1:2], pos[:, 2:3]
    d2g = ((px - gpT[0:1, :]) ** 2 + (py - gpT[1:2, :]) ** 2
           + (pz - gpT[2:3, :]) ** 2)
    jlt27 = (jax.lax.broadcasted_iota(jnp.int32, (27, 27), 0)
             < jax.lax.broadcasted_iota(jnp.int32, (27, 27), 1)).astype(f32)
    cmp27 = jnp.where(d2g[:, :, None] < d2g[:, None, :], 1.0,
                      jnp.where(d2g[:, :, None] == d2g[:, None, :],
                                jlt27[None], 0.0))
    rank27 = jnp.sum(cmp27, axis=1)
    w = jnp.where(rank27 < 3.0, 1.0 / jnp.clip(d2g, 1e-16, None), 0.0)
    x = _dot3(w, grid) / jnp.sum(w, axis=1, keepdims=True)
    x = jax.nn.gelu(_ln(x, ln2g_ref[...], ln2b_ref[...]))

    offs3 = offs_ref[...].reshape(1, 1, R)
    jlt = (jax.lax.broadcasted_iota(jnp.int32, (NPG, NPG), 0)
           < jax.lax.broadcasted_iota(jnp.int32, (NPG, NPG), 1)).astype(f32)
    ridx = jax.lax.broadcasted_iota(jnp.int32, (NPG, NPG), 0)
    cidx = jax.lax.broadcasted_iota(jnp.int32, (NPG, NPG), 1)

    for l in range(L):
        px, py, pz = pos[:, 0:1], pos[:, 1:2], pos[:, 2:3]
        pd2 = ((px - _rowvec(px)) ** 2 + (py - _rowvec(py)) ** 2
               + (pz - _rowvec(pz)) ** 2)
        pd2 = jnp.where(ridx == cidx, pd2 + 1e9, pd2)

        keys = jax.lax.bitcast_convert_type(pd2, jnp.int32)

        def _bs_body(_, lohi):
            lo, hi = lohi
            mid = lo + (hi - lo) // 2
            cle = jnp.sum((keys <= mid).astype(jnp.int32), axis=1,
                          keepdims=True)
            ge = cle >= KNB
            return (jnp.where(ge, lo, mid + 1), jnp.where(ge, mid, hi))

        lo0 = jnp.zeros((NPG, 1), jnp.int32)
        hi0 = jnp.full((NPG, 1), 0x7f800000, jnp.int32)
        tau, _ = jax.lax.fori_loop(0, 31, _bs_body, (lo0, hi0))
        clt = jnp.sum((keys < tau).astype(jnp.int32), axis=1, keepdims=True)
        need = (KNB - clt).astype(jnp.float32)
        eqf = (keys == tau).astype(jnp.float32)
        eqrank = jnp.dot(eqf, jlt, precision=jax.lax.Precision.DEFAULT)
        sel = jnp.where(((keys < tau) | ((eqf > 0.0) & (eqrank < need)))
                        & (pd2 < CUTOFF ** 2), 1.0, 0.0)

        q = _dot3(x, Wq_ref[l])
        kf = _dot3(x, Wk_ref[l])
        vf = _dot3(x, Wv_ref[l])
        sk = _dot3(x, Wsk_ref[l])
        We_l = We_ref[l]

        chunks = []
        for ic in range(NPG // _CI):
            sl_q = slice(ic * _CI, (ic + 1) * _CI)
            q_c = q[sl_q]
            pd2_c = pd2[sl_q]
            sel_c = sel[sl_q]
            dist = jnp.sqrt(jnp.clip(pd2_c, 1e-12, None))
            rbf = jnp.exp2((_COEFF * _LOG2E) * (dist[:, :, None] - offs3) ** 2)
            ee = jax.lax.dot_general(
                rbf.reshape(_CI * NPG, R), We_l, (((1,), (0,)), ((), ())),
                precision=_DEF).reshape(_CI, NPG, D)
            hsel = (jax.lax.broadcasted_iota(jnp.int32, (D, H), 0) // DH
                    == jax.lax.broadcasted_iota(jnp.int32, (D, H), 1)
                    ).astype(f32)
            prod = q_c[:, None, :] * ee
            eterm_all = jax.lax.dot_general(
                prod.reshape(_CI * NPG, D), hsel, (((1,), (0,)), ((), ())),
                precision=_DEF).reshape(_CI, NPG, H)
            heads = []
            for h in range(H):
                hs = slice(h * DH, (h + 1) * DH)
                qh = q_c[:, hs]
                core = _dot3(qh, kf[:, hs], (((1,), (1,)), ((), ())))
                lg = (core + eterm_all[:, :, h]) / _SQRT_DH
                lgm = jnp.where(sel_c > 0.0, lg, -1e9)
                m = jnp.max(lgm, axis=1, keepdims=True)
                p = jnp.exp(lgm - m) * sel_c
                den = jnp.sum(p, axis=1, keepdims=True)
                alpha = p / jnp.maximum(den, 1e-30)
                vh = _dot3(alpha, vf[:, hs])
                evh = jnp.sum(alpha[:, :, None] * ee[:, :, hs], axis=1)
                heads.append(vh + evh)
            chunks.append(jnp.concatenate(heads, axis=1))
        attn = jnp.concatenate(chunks, axis=0)

        x = _ln(attn + sk, lnmg_ref[l], lnmb_ref[l])
        for f in range(2):
            x = jax.nn.gelu(_ln(
                _dot3(x, Wm_ref[l, f]) + bm_ref[l, f],
                mg_ref[l, f], mb_ref[l, f]))
        pos = pos + x[:, 0:3]
        x = jnp.concatenate([x[:, 0:D - 3], pos], axis=1)

    logits_out = _dot3(x, Wo_ref[...]) + bo_ref[...]
    out_ref[...] = jnp.concatenate([pos, logits_out], axis=1)


def kernel(encoding, pos, batch, graph_sizes, W_up, b_up, ln1_g, ln1_b,
           ln2_g, ln2_b, Wq, Wk, Wv, Wskip, We, lnm_g, lnm_b, W_mlp, b_mlp,
           mlp_g, mlp_b, W_out, b_out):
    W2 = W_up.reshape(D, D, 27).transpose(0, 2, 1).reshape(D, 27 * D)
    b2 = b_up.reshape(D, 27).transpose(1, 0).reshape(1, 27 * D)
    gpT = jnp.asarray(_grid_positions_T())
    offs = jnp.asarray(_OFFS.reshape(1, R))
    ln1g = ln1_g.reshape(1, D); ln1b = ln1_b.reshape(1, D)
    ln2g = ln2_g.reshape(1, D); ln2b = ln2_b.reshape(1, D)
    lnmg = lnm_g.reshape(L, 1, D); lnmb = lnm_b.reshape(L, 1, D)
    bm = b_mlp.reshape(L, 2, 1, D)
    mg = mlp_g.reshape(L, 2, 1, D); mb = mlp_b.reshape(L, 2, 1, D)
    bo = b_out.reshape(1, NTYPES)

    def full(a):
        nd = a.ndim
        return pl.BlockSpec(a.shape, lambda b, _n=nd: (0,) * _n)

    enc3 = encoding.reshape(B, 1, D)
    args = (enc3, pos, gpT, offs, W2, b2, ln1g, ln1b, ln2g, ln2b,
            Wq, Wk, Wv, Wskip, We, lnmg, lnmb, W_mlp, bm, mg, mb, W_out, bo)
    in_specs = [
        pl.BlockSpec((1, 1, D), lambda b: (b, 0, 0)),
        pl.BlockSpec((NPG, 3), lambda b: (b, 0)),
    ] + [full(a) for a in args[2:]]

    out = pl.pallas_call(
        _fwd_kernel,
        grid=(B,),
        in_specs=in_specs,
        out_specs=pl.BlockSpec((NPG, 3 + NTYPES), lambda b: (b, 0)),
        out_shape=jax.ShapeDtypeStruct((N, 3 + NTYPES), jnp.float32),
        compiler_params=pltpu.CompilerParams(
            dimension_semantics=("parallel",)),
    )(*args)
    return out


# speedup vs baseline: 3.4963x; 1.0192x over previous
import numpy as np
import jax
import jax.numpy as jnp
from jax.experimental import pallas as pl
from jax.experimental.pallas import tpu as pltpu

B = 4
NPG = 256
N = B * NPG
D = 128
H = 4
DH = D // H
L = 2
R = 50
CUTOFF = 2.0
KNB = 100
NTYPES = 10

_CI = 128
_HI = jax.lax.Precision.HIGHEST
_DEF = jax.lax.Precision.DEFAULT


def _dot3(a, b, dims=None):
    if dims is None:
        dims = (((a.ndim - 1,), (0,)), ((), ()))
    ah = a.astype(jnp.bfloat16).astype(jnp.float32)
    al = a - ah
    bh = b.astype(jnp.bfloat16).astype(jnp.float32)
    bl = b - bh

    def d(x, y):
        return jax.lax.dot_general(x, y, dims, precision=_DEF)

    return d(ah, bh) + d(al, bh) + d(ah, bl)

_OFFS = np.linspace(0.0, CUTOFF, R).astype(np.float32)
_COEFF = float(-0.5 / (_OFFS[1] - _OFFS[0]) ** 2)
_SQRT_DH = float(np.sqrt(np.float32(DH)))
_LOG2E = float(np.log2(np.exp(np.float64(1.0))))


def _grid_positions_T():
    g = np.zeros((27, 3), dtype=np.float32)
    i = 0
    for xx in range(-1, 2):
        for yy in range(-1, 2):
            for zz in range(-1, 2):
                g[i] = (xx, yy, zz)
                i += 1
    return g.T.copy()


def _ln(x, g, b, eps=1e-5):
    m = x.mean(-1, keepdims=True)
    v = ((x - m) ** 2).mean(-1, keepdims=True)
    return (x - m) / jnp.sqrt(v + eps) * g + b


def _rowvec(v):
    n = v.shape[0]
    eye = (jax.lax.broadcasted_iota(jnp.int32, (n, n), 0)
           == jax.lax.broadcasted_iota(jnp.int32, (n, n), 1)).astype(jnp.float32)
    return jax.lax.dot_general(v, eye, (((0,), (0,)), ((), ())), precision=_HI)


def _fwd_kernel(enc_ref, pos_ref, gpT_ref, offs_ref, W2_ref, b2_ref,
                ln1g_ref, ln1b_ref, ln2g_ref, ln2b_ref,
                Wq_ref, Wk_ref, Wv_ref, Wsk_ref, We_ref,
                lnmg_ref, lnmb_ref, Wm_ref, bm_ref, mg_ref, mb_ref,
                Wo_ref, bo_ref, out_ref):
    f32 = jnp.float32

    enc = enc_ref[...].reshape(1, D)
    grid = (_dot3(enc, W2_ref[...]) + b2_ref[...]).reshape(27, D)
    grid = jax.nn.gelu(_ln(grid, ln1g_ref[...], ln1b_ref[...]))

    pos = pos_ref[...]
    gpT = gpT_ref[...]

    px, py, pz = pos[:, 0:1], pos[:, 1:2], pos[:, 2:3]
    d2g = ((px - gpT[0:1, :]) ** 2 + (py - gpT[1:2, :]) ** 2
           + (pz - gpT[2:3, :]) ** 2)
    jlt27 = (jax.lax.broadcasted_iota(jnp.int32, (27, 27), 0)
             < jax.lax.broadcasted_iota(jnp.int32, (27, 27), 1)).astype(f32)
    cmp27 = jnp.where(d2g[:, :, None] < d2g[:, None, :], 1.0,
                      jnp.where(d2g[:, :, None] == d2g[:, None, :],
                                jlt27[None], 0.0))
    rank27 = jnp.sum(cmp27, axis=1)
    w = jnp.where(rank27 < 3.0, 1.0 / jnp.clip(d2g, 1e-16, None), 0.0)
    x = _dot3(w, grid) / jnp.sum(w, axis=1, keepdims=True)
    x = jax.nn.gelu(_ln(x, ln2g_ref[...], ln2b_ref[...]))

    offs3 = offs_ref[...].reshape(1, 1, R)
    jlt = (jax.lax.broadcasted_iota(jnp.int32, (NPG, NPG), 0)
           < jax.lax.broadcasted_iota(jnp.int32, (NPG, NPG), 1)).astype(f32)
    ridx = jax.lax.broadcasted_iota(jnp.int32, (NPG, NPG), 0)
    cidx = jax.lax.broadcasted_iota(jnp.int32, (NPG, NPG), 1)

    for l in range(L):
        px, py, pz = pos[:, 0:1], pos[:, 1:2], pos[:, 2:3]
        pd2 = ((px - _rowvec(px)) ** 2 + (py - _rowvec(py)) ** 2
               + (pz - _rowvec(pz)) ** 2)
        pd2 = jnp.where(ridx == cidx, pd2 + 1e9, pd2)

        keys = jax.lax.bitcast_convert_type(pd2, jnp.int32)

        def _bs_body(_, lohi):
            lo, hi = lohi
            mid = lo + (hi - lo) // 2
            cle = jnp.sum((keys <= mid).astype(jnp.int32), axis=1,
                          keepdims=True)
            ge = cle >= KNB
            return (jnp.where(ge, lo, mid + 1), jnp.where(ge, mid, hi))

        lo0 = jnp.zeros((NPG, 1), jnp.int32)
        hi0 = jnp.full((NPG, 1), 0x7f800000, jnp.int32)
        tau, _ = jax.lax.fori_loop(0, 31, _bs_body, (lo0, hi0))
        clt = jnp.sum((keys < tau).astype(jnp.int32), axis=1, keepdims=True)
        need = (KNB - clt).astype(jnp.float32)
        eqf = (keys == tau).astype(jnp.float32)
        eqrank = jnp.dot(eqf, jlt, precision=jax.lax.Precision.DEFAULT)
        sel = jnp.where(((keys < tau) | ((eqf > 0.0) & (eqrank < need)))
                        & (pd2 < CUTOFF ** 2), 1.0, 0.0)

        q = _dot3(x, Wq_ref[l])
        kf = _dot3(x, Wk_ref[l])
        vf = _dot3(x, Wv_ref[l])
        sk = _dot3(x, Wsk_ref[l])
        We_l = We_ref[l]

        chunks = []
        for ic in range(NPG // _CI):
            sl_q = slice(ic * _CI, (ic + 1) * _CI)
            q_c = q[sl_q]
            pd2_c = pd2[sl_q]
            sel_c = sel[sl_q]
            dist = jnp.sqrt(jnp.clip(pd2_c, 1e-12, None))
            rbf = jnp.exp2((_COEFF * _LOG2E) * (dist[:, :, None] - offs3) ** 2)
            ee = jax.lax.dot_general(
                rbf.reshape(_CI * NPG, R), We_l, (((1,), (0,)), ((), ())),
                precision=_DEF).reshape(_CI, NPG, D)
            hsel = (jax.lax.broadcasted_iota(jnp.int32, (D, H), 0) // DH
                    == jax.lax.broadcasted_iota(jnp.int32, (D, H), 1)
                    ).astype(f32)
            prod = q_c[:, None, :] * ee
            eterm_all = jax.lax.dot_general(
                prod.reshape(_CI * NPG, D), hsel, (((1,), (0,)), ((), ())),
                precision=_DEF).reshape(_CI, NPG, H)
            heads = []
            for h in range(H):
                hs = slice(h * DH, (h + 1) * DH)
                qh = q_c[:, hs]
                core = _dot3(qh, kf[:, hs], (((1,), (1,)), ((), ())))
                lg = (core + eterm_all[:, :, h]) / _SQRT_DH
                lgm = jnp.where(sel_c > 0.0, lg, -1e9)
                m = jnp.max(lgm, axis=1, keepdims=True)
                p = jnp.exp2((lgm - m) * _LOG2E) * sel_c
                den = jnp.sum(p, axis=1, keepdims=True)
                alpha = p / jnp.maximum(den, 1e-30)
                vh = _dot3(alpha, vf[:, hs])
                evh = jnp.sum(alpha[:, :, None] * ee[:, :, hs], axis=1)
                heads.append(vh + evh)
            chunks.append(jnp.concatenate(heads, axis=1))
        attn = jnp.concatenate(chunks, axis=0)

        x = _ln(attn + sk, lnmg_ref[l], lnmb_ref[l])
        for f in range(2):
            x = jax.nn.gelu(_ln(
                _dot3(x, Wm_ref[l, f]) + bm_ref[l, f],
                mg_ref[l, f], mb_ref[l, f]))
        pos = pos + x[:, 0:3]
        x = jnp.concatenate([x[:, 0:D - 3], pos], axis=1)

    logits_out = _dot3(x, Wo_ref[...]) + bo_ref[...]
    out_ref[...] = jnp.concatenate([pos, logits_out], axis=1)


def kernel(encoding, pos, batch, graph_sizes, W_up, b_up, ln1_g, ln1_b,
           ln2_g, ln2_b, Wq, Wk, Wv, Wskip, We, lnm_g, lnm_b, W_mlp, b_mlp,
           mlp_g, mlp_b, W_out, b_out):
    W2 = W_up.reshape(D, D, 27).transpose(0, 2, 1).reshape(D, 27 * D)
    b2 = b_up.reshape(D, 27).transpose(1, 0).reshape(1, 27 * D)
    gpT = jnp.asarray(_grid_positions_T())
    offs = jnp.asarray(_OFFS.reshape(1, R))
    ln1g = ln1_g.reshape(1, D); ln1b = ln1_b.reshape(1, D)
    ln2g = ln2_g.reshape(1, D); ln2b = ln2_b.reshape(1, D)
    lnmg = lnm_g.reshape(L, 1, D); lnmb = lnm_b.reshape(L, 1, D)
    bm = b_mlp.reshape(L, 2, 1, D)
    mg = mlp_g.reshape(L, 2, 1, D); mb = mlp_b.reshape(L, 2, 1, D)
    bo = b_out.reshape(1, NTYPES)

    def full(a):
        nd = a.ndim
        return pl.BlockSpec(a.shape, lambda b, _n=nd: (0,) * _n)

    enc3 = encoding.reshape(B, 1, D)
    args = (enc3, pos, gpT, offs, W2, b2, ln1g, ln1b, ln2g, ln2b,
            Wq, Wk, Wv, Wskip, We, lnmg, lnmb, W_mlp, bm, mg, mb, W_out, bo)
    in_specs = [
        pl.BlockSpec((1, 1, D), lambda b: (b, 0, 0)),
        pl.BlockSpec((NPG, 3), lambda b: (b, 0)),
    ] + [full(a) for a in args[2:]]

    out = pl.pallas_call(
        _fwd_kernel,
        grid=(B,),
        in_specs=in_specs,
        out_specs=pl.BlockSpec((NPG, 3 + NTYPES), lambda b: (b, 0)),
        out_shape=jax.ShapeDtypeStruct((N, 3 + NTYPES), jnp.float32),
        compiler_params=pltpu.CompilerParams(
            dimension_semantics=("parallel",)),
    )(*args)
    return out
